# Initial kernel scaffold; baseline (speedup 1.0000x reference)
import jax
import jax.numpy as jnp
from jax.experimental import pallas as pl


def kernel(x, edge_index, batch, W1_0, b1_0, W2_0, b2_0, eps_0, W1_1, b1_1, W2_1, b2_1, eps_1, W1_2, b1_2, W2_2, b2_2, eps_2, Wf1, bf1, Wf2, bf2):
    raise NotImplementedError("write your pallas kernel here")


# speedup vs baseline: 2.7585x; 2.7585x over previous
import functools

import jax
import jax.numpy as jnp
from jax import lax
from jax.experimental import pallas as pl
from jax.experimental.pallas import tpu as pltpu
from jax.experimental.pallas import tpu_sc as plsc

NW = 32
NS = 16
C = 128
IB = 40
ZR = 128


@functools.lru_cache(maxsize=None)
def _seg_sum_kernel(n_blocks, acc_r, F):
    rpt = acc_r // NS
    n_z = rpt // ZR

    mesh = plsc.VectorSubcoreMesh(core_axis_name="c", subcore_axis_name="s")

    @functools.partial(
        pl.kernel,
        mesh=mesh,
        out_type=jax.ShapeDtypeStruct((2, acc_r, F), jnp.float32),
        scratch_types=[
            pltpu.VMEM((IB, C), jnp.int32),
            pltpu.VMEM((IB, C), jnp.int32),
            pltpu.VMEM((C, F), jnp.float32),
            pltpu.VMEM((ZR, F), jnp.float32),
            pltpu.VMEM_SHARED((acc_r, F), jnp.float32),
            pltpu.SemaphoreType.DMA,
        ],
    )
    def seg_sum(x_hbm, srcm_hbm, dstm_hbm, z_hbm, out_hbm,
                src_v, dst_v, rows_v, zero_v, acc_sh, sem):
        core = lax.axis_index("c")
        sub = lax.axis_index("s")
        wid = sub * 2 + core

        pltpu.sync_copy(z_hbm, zero_v)
        for k in range(n_z):
            pltpu.sync_copy(zero_v, acc_sh.at[pl.ds(sub * rpt + k * ZR, ZR)])
        plsc.subcore_barrier()

        def block(b, carry):
            pltpu.sync_copy(srcm_hbm.at[wid, pl.ds(b * IB, IB)], src_v)
            pltpu.sync_copy(dstm_hbm.at[wid, pl.ds(b * IB, IB)], dst_v)

            def chunk(j, c2):
                pltpu.async_copy(x_hbm.at[src_v.at[j]], rows_v, sem).wait()
                pltpu.sync_copy(rows_v, acc_sh.at[dst_v.at[j]], add=True)
                return c2

            lax.fori_loop(0, IB, chunk, 0)
            return carry

        lax.fori_loop(0, n_blocks, block, 0)
        plsc.subcore_barrier()

        for k in range(n_z):
            r0 = sub * rpt + k * ZR
            pltpu.sync_copy(acc_sh.at[pl.ds(r0, ZR)], zero_v)
            pltpu.sync_copy(zero_v, out_hbm.at[core, pl.ds(r0, ZR)])

    return seg_sum


def _mlp_body(eps_ref, x_ref, p0_ref, p1_ref, w1_ref, b1_ref, w2_ref, b2_ref,
              o_ref):
    h = (1.0 + eps_ref[0, 0]) * x_ref[...] + p0_ref[0] + p1_ref[0]
    r = jnp.maximum(
        jnp.dot(h, w1_ref[...], preferred_element_type=jnp.float32)
        + b1_ref[...], 0.0)
    o_ref[...] = (jnp.dot(r, w2_ref[...], preferred_element_type=jnp.float32)
                  + b2_ref[...])


@functools.lru_cache(maxsize=None)
def _mlp_kernel(N, F, R, acc_r):
    grid = N // R
    return pl.pallas_call(
        _mlp_body,
        grid=(grid,),
        in_specs=[
            pl.BlockSpec((1, 1), lambda i: (0, 0)),
            pl.BlockSpec((R, F), lambda i: (i, 0)),
            pl.BlockSpec((1, R, F), lambda i: (0, i, 0)),
            pl.BlockSpec((1, R, F), lambda i: (1, i, 0)),
            pl.BlockSpec((F, F), lambda i: (0, 0)),
            pl.BlockSpec((1, F), lambda i: (0, 0)),
            pl.BlockSpec((F, F), lambda i: (0, 0)),
            pl.BlockSpec((1, F), lambda i: (0, 0)),
        ],
        out_specs=pl.BlockSpec((R, F), lambda i: (i, 0)),
        out_shape=jax.ShapeDtypeStruct((N, F), jnp.float32),
    )


@functools.lru_cache(maxsize=None)
def _l3_kernel(N, F, R, acc_r, G, HF, T):
    grid = N // R

    def body(eps_ref, x_ref, p0_ref, p1_ref, b_ref, w1_ref, b1_ref, w2_ref,
             b2_ref, wf1_ref, bf1_ref, wf2_ref, bf2_ref, o_ref, pp_ref,
             cnt_ref):
        i = pl.program_id(0)

        @pl.when(i == 0)
        def _():
            pp_ref[...] = jnp.zeros_like(pp_ref)
            cnt_ref[...] = jnp.zeros_like(cnt_ref)

        h = (1.0 + eps_ref[0, 0]) * x_ref[...] + p0_ref[0] + p1_ref[0]
        r = jnp.maximum(
            jnp.dot(h, w1_ref[...], preferred_element_type=jnp.float32)
            + b1_ref[...], 0.0)
        gi = lax.broadcasted_iota(jnp.int32, (G, R), 0)
        oh = (gi == b_ref[0, 0, :][None, :]).astype(jnp.float32)
        pp_ref[...] += jnp.dot(oh, r, preferred_element_type=jnp.float32)
        cnt_ref[...] += jnp.broadcast_to(
            jnp.sum(oh, axis=1, keepdims=True), (G, F))

        @pl.when(i == grid - 1)
        def _():
            pooled = (jnp.dot(pp_ref[...], w2_ref[...],
                              preferred_element_type=jnp.float32)
                      + cnt_ref[...] * b2_ref[...])
            f = jnp.maximum(
                jnp.dot(pooled, wf1_ref[...],
                        preferred_element_type=jnp.float32) + bf1_ref[...],
                0.0)
            o_ref[...] = (jnp.dot(f, wf2_ref[...],
                                  preferred_element_type=jnp.float32)
                          + bf2_ref[...])

    return pl.pallas_call(
        body,
        grid=(grid,),
        in_specs=[
            pl.BlockSpec((1, 1), lambda i: (0, 0)),
            pl.BlockSpec((R, F), lambda i: (i, 0)),
            pl.BlockSpec((1, R, F), lambda i: (0, i, 0)),
            pl.BlockSpec((1, R, F), lambda i: (1, i, 0)),
            pl.BlockSpec((1, 1, R), lambda i: (i, 0, 0)),
            pl.BlockSpec((F, F), lambda i: (0, 0)),
            pl.BlockSpec((1, F), lambda i: (0, 0)),
            pl.BlockSpec((F, F), lambda i: (0, 0)),
            pl.BlockSpec((1, F), lambda i: (0, 0)),
            pl.BlockSpec((F, HF), lambda i: (0, 0)),
            pl.BlockSpec((1, HF), lambda i: (0, 0)),
            pl.BlockSpec((HF, T), lambda i: (0, 0)),
            pl.BlockSpec((1, T), lambda i: (0, 0)),
        ],
        out_specs=pl.BlockSpec((G, T), lambda i: (0, 0)),
        out_shape=jax.ShapeDtypeStruct((G, T), jnp.float32),
        scratch_shapes=[
            pltpu.VMEM((G, F), jnp.float32),
            pltpu.VMEM((G, F), jnp.float32),
        ],
    )


def kernel(x, edge_index, batch,
           W1_0, b1_0, W2_0, b2_0, eps_0,
           W1_1, b1_1, W2_1, b2_1, eps_1,
           W1_2, b1_2, W2_2, b2_2, eps_2,
           Wf1, bf1, Wf2, bf2):
    N, F = x.shape
    E = edge_index.shape[1]
    G = 32
    HF = Wf1.shape[1]
    T = Wf2.shape[1]

    n_blocks = -(-E // (NW * IB * C))
    e_pad = NW * n_blocks * IB * C
    acc_r = -(-(N + 1) // (NS * ZR)) * (NS * ZR)

    src = edge_index[0]
    dst = edge_index[1]
    pad = e_pad - E
    srcm = jnp.concatenate(
        [src, jnp.zeros((pad,), jnp.int32)]).reshape(NW, n_blocks * IB, C)
    dstm = jnp.concatenate(
        [dst, jnp.full((pad,), acc_r - 1, jnp.int32)]
    ).reshape(NW, n_blocks * IB, C)
    z = jnp.zeros((ZR, F), jnp.float32)

    seg = _seg_sum_kernel(n_blocks, acc_r, F)
    R = 2000
    mlp = _mlp_kernel(N, F, R, acc_r)
    l3 = _l3_kernel(N, F, R, acc_r, G, HF, T)
    batch_r = batch.reshape(N // R, 1, R)

    h = x
    for (W1, b1, W2, b2, eps) in (
            (W1_0, b1_0, W2_0, b2_0, eps_0),
            (W1_1, b1_1, W2_1, b2_1, eps_1)):
        parts = seg(h, srcm, dstm, z)
        h = mlp(eps.reshape(1, 1), h, parts, parts,
                W1, b1.reshape(1, F), W2, b2.reshape(1, F))

    parts = seg(h, srcm, dstm, z)
    out = l3(eps_2.reshape(1, 1), h, parts, parts, batch_r,
             W1_2, b1_2.reshape(1, F), W2_2, b2_2.reshape(1, F),
             Wf1, bf1.reshape(1, HF), Wf2, bf2.reshape(1, T))
    return out


# speedup vs baseline: 10.7627x; 3.9017x over previous
import functools

import jax
import jax.numpy as jnp
from jax import lax
from jax.experimental import pallas as pl
from jax.experimental.pallas import tpu as pltpu
from jax.experimental.pallas import tpu_sc as plsc

NW = 32
NS = 16
C = 120
IB = 4
K = 2


@functools.lru_cache(maxsize=None)
def _seg_sum_kernel(n_blocks, acc_r, F):
    rpt = acc_r // NS

    mesh = plsc.VectorSubcoreMesh(core_axis_name="c", subcore_axis_name="s")

    @functools.partial(
        pl.kernel,
        mesh=mesh,
        out_type=jax.ShapeDtypeStruct((2, acc_r, F), jnp.float32),
        scratch_types=[
            pltpu.VMEM((2, IB, C), jnp.int32),
            pltpu.VMEM((2, IB, C), jnp.int32),
            pltpu.VMEM((K, C, F), jnp.float32),
            pltpu.VMEM_SHARED((acc_r, F), jnp.float32),
            pltpu.SemaphoreType.DMA,
            pltpu.SemaphoreType.DMA,
            pltpu.SemaphoreType.DMA,
        ],
    )
    def seg_sum(x_hbm, srcm_hbm, dstm_hbm, z_hbm, out_hbm,
                src_v, dst_v, rows_v, acc_sh, gsem, ssem, isem):
        core = lax.axis_index("c")
        sub = lax.axis_index("s")
        wid = sub * 2 + core

        def wait_g(sl, j, b):
            pltpu.make_async_copy(x_hbm.at[src_v.at[sl, j]], rows_v.at[b],
                                  gsem).wait()

        def wait_s(sl, j, b):
            pltpu.make_async_copy(rows_v.at[b], acc_sh.at[dst_v.at[sl, j]],
                                  ssem).wait()

        def wait_i(blk, sl):
            pltpu.make_async_copy(srcm_hbm.at[wid, blk], src_v.at[sl],
                                  isem).wait()
            pltpu.make_async_copy(dstm_hbm.at[wid, blk], dst_v.at[sl],
                                  isem).wait()

        def fetch_i(blk, sl):
            pltpu.async_copy(srcm_hbm.at[wid, blk], src_v.at[sl], isem)
            pltpu.async_copy(dstm_hbm.at[wid, blk], dst_v.at[sl], isem)

        zcp = pltpu.make_async_copy(z_hbm, acc_sh.at[pl.ds(sub * rpt, rpt)],
                                    ssem)
        zcp.start()

        pltpu.sync_copy(srcm_hbm.at[wid, 0], src_v.at[0])
        pltpu.sync_copy(dstm_hbm.at[wid, 0], dst_v.at[0])
        if n_blocks > 1:
            fetch_i(1, 1)
        for b in range(K):
            pltpu.async_copy(x_hbm.at[src_v.at[0, b]], rows_v.at[b], gsem)
        zcp.wait()
        plsc.subcore_barrier()

        n_pairs = n_blocks * IB // K
        ppb = IB // K

        def pair(t, carry):
            blk = t // ppb
            sl = lax.rem(blk, 2)
            jb = lax.rem(t, ppb) * K
            for b in range(K):
                wait_g(sl, jb + b, b)
                pltpu.async_copy(rows_v.at[b],
                                 acc_sh.at[dst_v.at[sl, jb + b]],
                                 ssem, add=True)

            @pl.when(jnp.logical_and(jb + K < IB, t + 1 < n_pairs))
            def _():
                for b in range(K):
                    wait_s(sl, jb + b, b)
                    pltpu.async_copy(x_hbm.at[src_v.at[sl, jb + K + b]],
                                     rows_v.at[b], gsem)

            @pl.when(jnp.logical_and(jb + K >= IB, t + 1 < n_pairs))
            def _():
                for b in range(K):
                    wait_s(sl, jb + b, b)

                @pl.when(blk + 2 < n_blocks)
                def _():
                    fetch_i(blk + 2, sl)

                wait_i(blk + 1, 1 - sl)
                for b in range(K):
                    pltpu.async_copy(x_hbm.at[src_v.at[1 - sl, b]],
                                     rows_v.at[b], gsem)

            @pl.when(t + 1 >= n_pairs)
            def _():
                for b in range(K):
                    wait_s(sl, jb + b, b)
            return carry

        lax.fori_loop(0, n_pairs, pair, 0)
        plsc.subcore_barrier()

        pltpu.sync_copy(acc_sh.at[pl.ds(sub * rpt, rpt)],
                        out_hbm.at[core, pl.ds(sub * rpt, rpt)])

    return seg_sum


def _mlp_body(eps_ref, x_ref, p0_ref, p1_ref, w1_ref, b1_ref, w2_ref, b2_ref,
              o_ref):
    h = (1.0 + eps_ref[0, 0]) * x_ref[...] + p0_ref[0] + p1_ref[0]
    r = jnp.maximum(
        jnp.dot(h, w1_ref[...], preferred_element_type=jnp.float32)
        + b1_ref[...], 0.0)
    o_ref[...] = (jnp.dot(r, w2_ref[...], preferred_element_type=jnp.float32)
                  + b2_ref[...])


@functools.lru_cache(maxsize=None)
def _mlp_kernel(N, F, R, acc_r):
    grid = N // R
    return pl.pallas_call(
        _mlp_body,
        grid=(grid,),
        in_specs=[
            pl.BlockSpec((1, 1), lambda i: (0, 0)),
            pl.BlockSpec((R, F), lambda i: (i, 0)),
            pl.BlockSpec((1, R, F), lambda i: (0, i, 0)),
            pl.BlockSpec((1, R, F), lambda i: (1, i, 0)),
            pl.BlockSpec((F, F), lambda i: (0, 0)),
            pl.BlockSpec((1, F), lambda i: (0, 0)),
            pl.BlockSpec((F, F), lambda i: (0, 0)),
            pl.BlockSpec((1, F), lambda i: (0, 0)),
        ],
        out_specs=pl.BlockSpec((R, F), lambda i: (i, 0)),
        out_shape=jax.ShapeDtypeStruct((N, F), jnp.float32),
    )


@functools.lru_cache(maxsize=None)
def _l3_kernel(N, F, R, acc_r, G, HF, T):
    grid = N // R

    def body(eps_ref, x_ref, p0_ref, p1_ref, b_ref, w1_ref, b1_ref, w2_ref,
             b2_ref, wf1_ref, bf1_ref, wf2_ref, bf2_ref, o_ref, pp_ref,
             cnt_ref):
        i = pl.program_id(0)

        @pl.when(i == 0)
        def _():
            pp_ref[...] = jnp.zeros_like(pp_ref)
            cnt_ref[...] = jnp.zeros_like(cnt_ref)

        h = (1.0 + eps_ref[0, 0]) * x_ref[...] + p0_ref[0] + p1_ref[0]
        r = jnp.maximum(
            jnp.dot(h, w1_ref[...], preferred_element_type=jnp.float32)
            + b1_ref[...], 0.0)
        gi = lax.broadcasted_iota(jnp.int32, (G, R), 0)
        oh = (gi == b_ref[0, 0, :][None, :]).astype(jnp.float32)
        pp_ref[...] += jnp.dot(oh, r, preferred_element_type=jnp.float32)
        cnt_ref[...] += jnp.broadcast_to(
            jnp.sum(oh, axis=1, keepdims=True), (G, F))

        @pl.when(i == grid - 1)
        def _():
            pooled = (jnp.dot(pp_ref[...], w2_ref[...],
                              preferred_element_type=jnp.float32)
                      + cnt_ref[...] * b2_ref[...])
            f = jnp.maximum(
                jnp.dot(pooled, wf1_ref[...],
                        preferred_element_type=jnp.float32) + bf1_ref[...],
                0.0)
            o_ref[...] = (jnp.dot(f, wf2_ref[...],
                                  preferred_element_type=jnp.float32)
                          + bf2_ref[...])

    return pl.pallas_call(
        body,
        grid=(grid,),
        in_specs=[
            pl.BlockSpec((1, 1), lambda i: (0, 0)),
            pl.BlockSpec((R, F), lambda i: (i, 0)),
            pl.BlockSpec((1, R, F), lambda i: (0, i, 0)),
            pl.BlockSpec((1, R, F), lambda i: (1, i, 0)),
            pl.BlockSpec((1, 1, R), lambda i: (i, 0, 0)),
            pl.BlockSpec((F, F), lambda i: (0, 0)),
            pl.BlockSpec((1, F), lambda i: (0, 0)),
            pl.BlockSpec((F, F), lambda i: (0, 0)),
            pl.BlockSpec((1, F), lambda i: (0, 0)),
            pl.BlockSpec((F, HF), lambda i: (0, 0)),
            pl.BlockSpec((1, HF), lambda i: (0, 0)),
            pl.BlockSpec((HF, T), lambda i: (0, 0)),
            pl.BlockSpec((1, T), lambda i: (0, 0)),
        ],
        out_specs=pl.BlockSpec((G, T), lambda i: (0, 0)),
        out_shape=jax.ShapeDtypeStruct((G, T), jnp.float32),
        scratch_shapes=[
            pltpu.VMEM((G, F), jnp.float32),
            pltpu.VMEM((G, F), jnp.float32),
        ],
    )


def kernel(x, edge_index, batch,
           W1_0, b1_0, W2_0, b2_0, eps_0,
           W1_1, b1_1, W2_1, b2_1, eps_1,
           W1_2, b1_2, W2_2, b2_2, eps_2,
           Wf1, bf1, Wf2, bf2):
    N, F = x.shape
    E = edge_index.shape[1]
    G = 32
    HF = Wf1.shape[1]
    T = Wf2.shape[1]

    n_blocks = -(-E // (NW * IB * C))
    e_pad = NW * n_blocks * IB * C
    acc_r = -(-(N + 1) // (NS * 8)) * (NS * 8)

    src = edge_index[0]
    dst = edge_index[1]
    pad = e_pad - E
    pad_src = jnp.arange(pad, dtype=jnp.int32) % N
    pad_dst = N + jnp.arange(pad, dtype=jnp.int32) % (acc_r - N)
    srcm = jnp.concatenate([src, pad_src]).reshape(NW, n_blocks, IB, C)
    dstm = jnp.concatenate([dst, pad_dst]).reshape(NW, n_blocks, IB, C)
    z = jnp.zeros((acc_r // NS, F), jnp.float32)

    seg = _seg_sum_kernel(n_blocks, acc_r, F)
    R = N
    mlp = _mlp_kernel(N, F, R, acc_r)
    l3 = _l3_kernel(N, F, R, acc_r, G, HF, T)
    batch_r = batch.reshape(N // R, 1, R)

    h = x
    for (W1, b1, W2, b2, eps) in (
            (W1_0, b1_0, W2_0, b2_0, eps_0),
            (W1_1, b1_1, W2_1, b2_1, eps_1)):
        parts = seg(h, srcm, dstm, z)
        h = mlp(eps.reshape(1, 1), h, parts, parts,
                W1, b1.reshape(1, F), W2, b2.reshape(1, F))

    parts = seg(h, srcm, dstm, z)
    out = l3(eps_2.reshape(1, 1), h, parts, parts, batch_r,
             W1_2, b1_2.reshape(1, F), W2_2, b2_2.reshape(1, F),
             Wf1, bf1.reshape(1, HF), Wf2, bf2.reshape(1, T))
    return out
